# baseline (speedup 1.0000x reference)
import functools

import jax
import jax.numpy as jnp
from jax import lax
from jax.experimental import pallas as pl
from jax.experimental.pallas import tpu as pltpu
from jax.experimental.pallas import tpu_sc as plsc

_NUM = {"a": 50000, "b": 50000, "g": 10000}
_RELS = [
    ("a", "a2b", "b", 150000), ("b", "b2a", "a", 150000),
    ("a", "a2g", "g", 30000), ("g", "g2a", "a", 30000),
    ("b", "b2g", "g", 30000), ("g", "g2b", "b", 30000),
    ("a", "a2a", "a", 80000), ("b", "b2b", "b", 80000),
    ("g", "g2g", "g", 10000),
]
_D = 128
_K = 128
_BLK = 256
_NTILE = 32
_EPAD = 2 * 16 * _K


def _pad_to(n, m):
    return ((n + m - 1) // m) * m


_NP = {t: _pad_to(n, _BLK) for t, n in _NUM.items()}
_NACC = {t: _pad_to(n + 8, 2048) for t, n in _NUM.items()}
_CHUNK = {"a": 10240, "b": 10240, "g": 10240}

@functools.lru_cache(maxsize=None)
def _get_mesh():
    return plsc.VectorSubcoreMesh(core_axis_name="c", subcore_axis_name="s")


@functools.lru_cache(maxsize=None)
def _scatter_kernel(n_z, n_acc, chunk_rows, e_pad):
    blocks = e_pad // (_NTILE * _K)
    n_chunks = n_acc // chunk_rows
    acc_rows = chunk_rows + _K
    rows_t = acc_rows // 16
    nfull = rows_t // _K
    rem = rows_t % _K
    out_t = chunk_rows // 16

    @functools.partial(
        pl.kernel,
        out_type=jax.ShapeDtypeStruct((2 * n_acc, _D), jnp.float32),
        mesh=_get_mesh(),
        scratch_types=[
            pltpu.VMEM((_K, _D), jnp.float32),
            pltpu.VMEM((_K,), jnp.int32),
            pltpu.VMEM((_K,), jnp.int32),
            pltpu.VMEM((_K, _D), jnp.float32),
            pltpu.VMEM_SHARED((acc_rows, _D), jnp.float32),
            pltpu.SemaphoreType.DMA,
        ],
    )
    def scat(src_hbm, lidx_hbm, z_hbm, zeros_hbm, m_hbm,
             zeros_v, sidx_v, lidx_v, rows_v, acc_sh, sem):
        cid = lax.axis_index("c")
        sid = lax.axis_index("s")
        pltpu.sync_copy(zeros_hbm, zeros_v)
        base = cid * (e_pad // 2) + sid * (e_pad // 32)

        for chunk in range(n_chunks):
            lo = chunk * chunk_rows
            lbase = chunk * e_pad + base

            def zero_acc(j, carry):
                pltpu.sync_copy(zeros_v,
                                acc_sh.at[pl.ds(sid * rows_t + j * _K, _K), :])
                return carry

            lax.fori_loop(0, nfull, zero_acc, 0)
            if rem:
                pltpu.sync_copy(
                    zeros_v.at[pl.ds(0, rem), :],
                    acc_sh.at[pl.ds(sid * rows_t + nfull * _K, rem), :])
            plsc.subcore_barrier()

            def edge_block(i, carry, lbase=lbase):
                off = base + i * _K
                pltpu.sync_copy(src_hbm.at[pl.ds(off, _K)], sidx_v)
                pltpu.sync_copy(lidx_hbm.at[pl.ds(lbase + i * _K, _K)], lidx_v)
                pltpu.async_copy(z_hbm.at[sidx_v], rows_v, sem).wait()
                pltpu.sync_copy(rows_v, acc_sh.at[lidx_v], add=True)
                return carry

            lax.fori_loop(0, blocks, edge_block, 0)
            plsc.subcore_barrier()
            pltpu.sync_copy(acc_sh.at[pl.ds(sid * out_t, out_t), :],
                            m_hbm.at[pl.ds(cid * n_acc + lo + sid * out_t, out_t), :])
            plsc.subcore_barrier()

    return scat


def _chunk_local_idx(dst, chunk_rows, n_chunks):
    e_pad = dst.shape[0]
    rows = e_pad // 512
    d2 = dst.reshape(rows, 512)

    def body(d_ref, o_ref):
        c = pl.program_id(0)
        lo = c * chunk_rows
        d = d_ref[...]
        inr = (d >= lo) & (d < lo + chunk_rows)
        o_ref[...] = jnp.where(inr, d - lo, chunk_rows)[None, :, :]

    out = pl.pallas_call(
        body,
        grid=(n_chunks,),
        in_specs=[pl.BlockSpec((rows, 512), lambda c: (0, 0))],
        out_specs=pl.BlockSpec((1, rows, 512), lambda c: (c, 0, 0)),
        out_shape=jax.ShapeDtypeStruct((n_chunks, rows, 512), jnp.int32),
    )(d2)
    return out.reshape(n_chunks * e_pad)


def _transform(x, d0, d1, w):
    n = x.shape[0]

    def body(x_ref, d0_ref, d1_ref, w_ref, o_ref):
        deg = d0_ref[...] + d1_ref[...]
        rs = lax.rsqrt(jnp.maximum(deg, 1.0))
        o_ref[...] = jnp.dot(x_ref[...] * rs, w_ref[...],
                             preferred_element_type=jnp.float32)

    return pl.pallas_call(
        body,
        grid=(n // _BLK,),
        in_specs=[
            pl.BlockSpec((_BLK, _D), lambda i: (i, 0)),
            pl.BlockSpec((_BLK, 1), lambda i: (i, 0)),
            pl.BlockSpec((_BLK, 1), lambda i: (i, 0)),
            pl.BlockSpec((_D, _D), lambda i: (0, 0)),
        ],
        out_specs=pl.BlockSpec((_BLK, _D), lambda i: (i, 0)),
        out_shape=jax.ShapeDtypeStruct((n, _D), jnp.float32),
    )(x, d0, d1, w)


def _combine(ms, dds, bsum, resid, n_p):
    flat = []
    for pair in ms:
        flat.extend(pair)
    for pair in dds:
        flat.extend(pair)
    flat.append(bsum)
    has_resid = resid is not None
    if has_resid:
        flat.append(resid)

    def body(*refs):
        o_ref = refs[-1]
        m_refs = refs[0:6]
        d_refs = refs[6:12]
        b_ref = refs[12]
        acc = jnp.broadcast_to(b_ref[...], (_BLK, _D))
        for r in range(3):
            deg = d_refs[2 * r][...] + d_refs[2 * r + 1][...]
            rs = lax.rsqrt(jnp.maximum(deg, 1.0))
            acc = acc + rs * (m_refs[2 * r][...] + m_refs[2 * r + 1][...])
        if has_resid:
            acc = acc + refs[13][...]
        o_ref[...] = acc

    in_specs = (
        [pl.BlockSpec((_BLK, _D), lambda i: (i, 0))] * 6
        + [pl.BlockSpec((_BLK, 1), lambda i: (i, 0))] * 6
        + [pl.BlockSpec((1, _D), lambda i: (0, 0))]
    )
    if has_resid:
        in_specs.append(pl.BlockSpec((_BLK, _D), lambda i: (i, 0)))

    return pl.pallas_call(
        body,
        grid=(n_p // _BLK,),
        in_specs=in_specs,
        out_specs=pl.BlockSpec((_BLK, _D), lambda i: (i, 0)),
        out_shape=jax.ShapeDtypeStruct((n_p, _D), jnp.float32),
    )(*flat)


def kernel(feat_a, feat_b, feat_g, edge_index_a2b, edge_index_b2a, edge_index_a2g, edge_index_g2a, edge_index_b2g, edge_index_g2b, edge_index_a2a, edge_index_b2b, edge_index_g2g, W0_a2b, b0_a2b, W0_b2a, b0_b2a, W0_a2g, b0_a2g, W0_g2a, b0_g2a, W0_b2g, b0_b2g, W0_g2b, b0_g2b, W0_a2a, b0_a2a, W0_b2b, b0_b2b, W0_g2g, b0_g2g, W1_a2b, b1_a2b, W1_b2a, b1_b2a, W1_a2g, b1_a2g, W1_g2a, b1_g2a, W1_b2g, b1_b2g, W1_g2b, b1_g2b, W1_a2a, b1_a2a, W1_b2b, b1_b2b, W1_g2g, b1_g2g):
    inp = dict(locals())
    feats0 = {t: inp["feat_" + t] for t in _NUM}

    fpad = {t: jnp.concatenate(
        [feats0[t], jnp.zeros((_NP[t] - _NUM[t], _D), jnp.float32)])
        for t in _NUM}

    zeros128 = jnp.zeros((_K, _D), jnp.float32)

    edges = {}
    for (s, et, d, ne) in _RELS:
        ei = inp["edge_index_" + et]
        e_pad = _pad_to(ne, _EPAD)
        src = jnp.concatenate(
            [ei[0], jnp.full((e_pad - ne,), _NUM[s], jnp.int32)])
        dst = jnp.concatenate(
            [ei[1], jnp.full((e_pad - ne,), _NUM[d], jnp.int32)])
        lidx = _chunk_local_idx(dst, _CHUNK[d], _NACC[d] // _CHUNK[d])
        edges[et] = (src, dst, lidx, e_pad)

    ones_tab = {t: jnp.ones((_NP[t], _D), jnp.float32) for t in _NUM}
    degs = {}
    for (s, et, d, ne) in _RELS:
        src, dst, lidx, e_pad = edges[et]
        lidx_s = _chunk_local_idx(src, _CHUNK[s], _NACC[s] // _CHUNK[s])
        hs = _scatter_kernel(_NP[s], _NACC[s], _CHUNK[s], e_pad)(
            src, lidx_s, ones_tab[s], zeros128).reshape(2, _NACC[s], _D)
        hd = _scatter_kernel(_NP[s], _NACC[d], _CHUNK[d], e_pad)(
            src, lidx, ones_tab[s], zeros128).reshape(2, _NACC[d], _D)
        degs[et] = (
            (hs[0, :_NP[s], 0:1], hs[1, :_NP[s], 0:1]),
            (hd[0, :_NP[d], 0:1], hd[1, :_NP[d], 0:1]),
        )

    feats = fpad
    for l in range(2):
        per_dst = {t: [] for t in _NUM}
        for (s, et, d, ne) in _RELS:
            src, dst, lidx, e_pad = edges[et]
            (ds0, ds1), (dd0, dd1) = degs[et]
            w = inp["W%d_%s" % (l, et)]
            z = _transform(feats[s], ds0, ds1, w)
            m = _scatter_kernel(_NP[s], _NACC[d], _CHUNK[d], e_pad)(
                src, lidx, z, zeros128)
            m = m.reshape(2, _NACC[d], _D)
            per_dst[d].append(((m[0], m[1]), (dd0, dd1), et))
        new = {}
        for t in _NUM:
            ms = [p[0] for p in per_dst[t]]
            dds = [p[1] for p in per_dst[t]]
            bsum = sum(inp["b%d_%s" % (l, p[2])] for p in per_dst[t])
            bsum = bsum.reshape(1, _D)
            resid = fpad[t] if l == 1 else None
            new[t] = _combine(ms, dds, bsum, resid, _NP[t])
        feats = new

    return (feats["a"][:_NUM["a"]], feats["b"][:_NUM["b"]],
            feats["g"][:_NUM["g"]])


# speedup vs baseline: 1.4553x; 1.4553x over previous
import functools

import jax
import jax.numpy as jnp
from jax import lax
from jax.experimental import pallas as pl
from jax.experimental.pallas import tpu as pltpu
from jax.experimental.pallas import tpu_sc as plsc

_NUM = {"a": 50000, "b": 50000, "g": 10000}
_RELS = [
    ("a", "a2b", "b", 150000), ("b", "b2a", "a", 150000),
    ("a", "a2g", "g", 30000), ("g", "g2a", "a", 30000),
    ("b", "b2g", "g", 30000), ("g", "g2b", "b", 30000),
    ("a", "a2a", "a", 80000), ("b", "b2b", "b", 80000),
    ("g", "g2g", "g", 10000),
]
_D = 128
_K = 128
_BLK = 256
_NTILE = 32
_EPAD = 2 * 16 * _K


def _pad_to(n, m):
    return ((n + m - 1) // m) * m


_NP = {t: _pad_to(n, _BLK) for t, n in _NUM.items()}
_NACC = {t: _pad_to(n + 8, 2048) for t, n in _NUM.items()}
_CHUNK = {"a": 10240, "b": 10240, "g": 10240}

@functools.lru_cache(maxsize=None)
def _get_mesh():
    return plsc.VectorSubcoreMesh(core_axis_name="c", subcore_axis_name="s")


@functools.lru_cache(maxsize=None)
def _scatter_kernel(n_z, n_acc, chunk_rows, e_pad):
    blocks = e_pad // (_NTILE * _K)
    n_chunks = n_acc // chunk_rows
    acc_rows = chunk_rows + _K
    rows_t = acc_rows // 16
    nfull = rows_t // _K
    rem = rows_t % _K
    out_t = chunk_rows // 16

    @functools.partial(
        pl.kernel,
        out_type=jax.ShapeDtypeStruct((2 * n_acc, _D), jnp.float32),
        mesh=_get_mesh(),
        scratch_types=[
            pltpu.VMEM((_K, _D), jnp.float32),
            pltpu.VMEM((_K,), jnp.int32),
            pltpu.VMEM((_K,), jnp.int32),
            pltpu.VMEM((_K, _D), jnp.float32),
            pltpu.VMEM_SHARED((acc_rows, _D), jnp.float32),
            pltpu.SemaphoreType.DMA,
        ],
    )
    def scat(src_hbm, lidx_hbm, z_hbm, zeros_hbm, m_hbm,
             zeros_v, sidx_v, lidx_v, rows_v, acc_sh, sem):
        cid = lax.axis_index("c")
        sid = lax.axis_index("s")
        pltpu.sync_copy(zeros_hbm, zeros_v)
        base = cid * (e_pad // 2) + sid * (e_pad // 32)

        for chunk in range(n_chunks):
            lo = chunk * chunk_rows
            lbase = chunk * e_pad + base

            def zero_acc(j, carry):
                pltpu.sync_copy(zeros_v,
                                acc_sh.at[pl.ds(sid * rows_t + j * _K, _K), :])
                return carry

            lax.fori_loop(0, nfull, zero_acc, 0)
            if rem:
                pltpu.sync_copy(
                    zeros_v.at[pl.ds(0, rem), :],
                    acc_sh.at[pl.ds(sid * rows_t + nfull * _K, rem), :])
            plsc.subcore_barrier()

            def edge_block(i, carry, lbase=lbase):
                off = base + i * _K
                pltpu.sync_copy(src_hbm.at[pl.ds(off, _K)], sidx_v)
                pltpu.sync_copy(lidx_hbm.at[pl.ds(lbase + i * _K, _K)], lidx_v)
                pltpu.async_copy(z_hbm.at[sidx_v], rows_v, sem).wait()
                pltpu.sync_copy(rows_v, acc_sh.at[lidx_v], add=True)
                return carry

            lax.fori_loop(0, blocks, edge_block, 0)
            plsc.subcore_barrier()
            pltpu.sync_copy(acc_sh.at[pl.ds(sid * out_t, out_t), :],
                            m_hbm.at[pl.ds(cid * n_acc + lo + sid * out_t, out_t), :])
            plsc.subcore_barrier()

    return scat


@functools.lru_cache(maxsize=None)
def _count_kernel(n_acc, chunk_rows, e_pad):
    blocks = e_pad // (_NTILE * _K)
    n_chunks = n_acc // chunk_rows
    acc_rows = chunk_rows + _K
    rows_t = acc_rows // 16
    nfull = rows_t // _K
    rem = rows_t % _K
    out_t = chunk_rows // 16

    @functools.partial(
        pl.kernel,
        out_type=jax.ShapeDtypeStruct((2 * n_acc, _D), jnp.float32),
        mesh=_get_mesh(),
        scratch_types=[
            pltpu.VMEM((_K, _D), jnp.float32),
            pltpu.VMEM((_K, _D), jnp.float32),
            pltpu.VMEM((_K,), jnp.int32),
            pltpu.VMEM_SHARED((acc_rows, _D), jnp.float32),
        ],
    )
    def count(lidx_hbm, zeros_hbm, ones_hbm, m_hbm,
              zeros_v, ones_v, lidx_v, acc_sh):
        cid = lax.axis_index("c")
        sid = lax.axis_index("s")
        pltpu.sync_copy(zeros_hbm, zeros_v)
        pltpu.sync_copy(ones_hbm, ones_v)
        base = cid * (e_pad // 2) + sid * (e_pad // 32)

        for chunk in range(n_chunks):
            lo = chunk * chunk_rows
            lbase = chunk * e_pad + base

            def zero_acc(j, carry):
                pltpu.sync_copy(zeros_v,
                                acc_sh.at[pl.ds(sid * rows_t + j * _K, _K), :])
                return carry

            lax.fori_loop(0, nfull, zero_acc, 0)
            if rem:
                pltpu.sync_copy(
                    zeros_v.at[pl.ds(0, rem), :],
                    acc_sh.at[pl.ds(sid * rows_t + nfull * _K, rem), :])
            plsc.subcore_barrier()

            def edge_block(i, carry, lbase=lbase):
                pltpu.sync_copy(lidx_hbm.at[pl.ds(lbase + i * _K, _K)], lidx_v)
                pltpu.sync_copy(ones_v, acc_sh.at[lidx_v], add=True)
                return carry

            lax.fori_loop(0, blocks, edge_block, 0)
            plsc.subcore_barrier()
            pltpu.sync_copy(acc_sh.at[pl.ds(sid * out_t, out_t), :],
                            m_hbm.at[pl.ds(cid * n_acc + lo + sid * out_t, out_t), :])
            plsc.subcore_barrier()

    return count


def _chunk_local_idx(dst, chunk_rows, n_chunks):
    e_pad = dst.shape[0]
    rows = e_pad // 512
    d2 = dst.reshape(rows, 512)

    def body(d_ref, o_ref):
        c = pl.program_id(0)
        lo = c * chunk_rows
        d = d_ref[...]
        inr = (d >= lo) & (d < lo + chunk_rows)
        o_ref[...] = jnp.where(inr, d - lo, chunk_rows)[None, :, :]

    out = pl.pallas_call(
        body,
        grid=(n_chunks,),
        in_specs=[pl.BlockSpec((rows, 512), lambda c: (0, 0))],
        out_specs=pl.BlockSpec((1, rows, 512), lambda c: (c, 0, 0)),
        out_shape=jax.ShapeDtypeStruct((n_chunks, rows, 512), jnp.int32),
    )(d2)
    return out.reshape(n_chunks * e_pad)


def _transform(x, d0, d1, w):
    n = x.shape[0]

    def body(x_ref, d0_ref, d1_ref, w_ref, o_ref):
        deg = d0_ref[...] + d1_ref[...]
        rs = lax.rsqrt(jnp.maximum(deg, 1.0))
        o_ref[...] = jnp.dot(x_ref[...] * rs, w_ref[...],
                             preferred_element_type=jnp.float32)

    return pl.pallas_call(
        body,
        grid=(n // _BLK,),
        in_specs=[
            pl.BlockSpec((_BLK, _D), lambda i: (i, 0)),
            pl.BlockSpec((_BLK, 1), lambda i: (i, 0)),
            pl.BlockSpec((_BLK, 1), lambda i: (i, 0)),
            pl.BlockSpec((_D, _D), lambda i: (0, 0)),
        ],
        out_specs=pl.BlockSpec((_BLK, _D), lambda i: (i, 0)),
        out_shape=jax.ShapeDtypeStruct((n, _D), jnp.float32),
    )(x, d0, d1, w)


def _combine(ms, dds, bsum, resid, n_p):
    flat = []
    for pair in ms:
        flat.extend(pair)
    for pair in dds:
        flat.extend(pair)
    flat.append(bsum)
    has_resid = resid is not None
    if has_resid:
        flat.append(resid)

    def body(*refs):
        o_ref = refs[-1]
        m_refs = refs[0:6]
        d_refs = refs[6:12]
        b_ref = refs[12]
        acc = jnp.broadcast_to(b_ref[...], (_BLK, _D))
        for r in range(3):
            deg = d_refs[2 * r][...] + d_refs[2 * r + 1][...]
            rs = lax.rsqrt(jnp.maximum(deg, 1.0))
            acc = acc + rs * (m_refs[2 * r][...] + m_refs[2 * r + 1][...])
        if has_resid:
            acc = acc + refs[13][...]
        o_ref[...] = acc

    in_specs = (
        [pl.BlockSpec((_BLK, _D), lambda i: (i, 0))] * 6
        + [pl.BlockSpec((_BLK, 1), lambda i: (i, 0))] * 6
        + [pl.BlockSpec((1, _D), lambda i: (0, 0))]
    )
    if has_resid:
        in_specs.append(pl.BlockSpec((_BLK, _D), lambda i: (i, 0)))

    return pl.pallas_call(
        body,
        grid=(n_p // _BLK,),
        in_specs=in_specs,
        out_specs=pl.BlockSpec((_BLK, _D), lambda i: (i, 0)),
        out_shape=jax.ShapeDtypeStruct((n_p, _D), jnp.float32),
    )(*flat)


def kernel(feat_a, feat_b, feat_g, edge_index_a2b, edge_index_b2a, edge_index_a2g, edge_index_g2a, edge_index_b2g, edge_index_g2b, edge_index_a2a, edge_index_b2b, edge_index_g2g, W0_a2b, b0_a2b, W0_b2a, b0_b2a, W0_a2g, b0_a2g, W0_g2a, b0_g2a, W0_b2g, b0_b2g, W0_g2b, b0_g2b, W0_a2a, b0_a2a, W0_b2b, b0_b2b, W0_g2g, b0_g2g, W1_a2b, b1_a2b, W1_b2a, b1_b2a, W1_a2g, b1_a2g, W1_g2a, b1_g2a, W1_b2g, b1_b2g, W1_g2b, b1_g2b, W1_a2a, b1_a2a, W1_b2b, b1_b2b, W1_g2g, b1_g2g):
    inp = dict(locals())
    feats0 = {t: inp["feat_" + t] for t in _NUM}

    fpad = {t: jnp.concatenate(
        [feats0[t], jnp.zeros((_NP[t] - _NUM[t], _D), jnp.float32)])
        for t in _NUM}

    zeros128 = jnp.zeros((_K, _D), jnp.float32)

    edges = {}
    for (s, et, d, ne) in _RELS:
        ei = inp["edge_index_" + et]
        e_pad = _pad_to(ne, _EPAD)
        src = jnp.concatenate(
            [ei[0], jnp.full((e_pad - ne,), _NUM[s], jnp.int32)])
        dst = jnp.concatenate(
            [ei[1], jnp.full((e_pad - ne,), _NUM[d], jnp.int32)])
        lidx = _chunk_local_idx(dst, _CHUNK[d], _NACC[d] // _CHUNK[d])
        edges[et] = (src, dst, lidx, e_pad)

    ones128 = jnp.ones((_K, _D), jnp.float32)
    degs = {}
    for (s, et, d, ne) in _RELS:
        src, dst, lidx, e_pad = edges[et]
        lidx_s = _chunk_local_idx(src, _CHUNK[s], _NACC[s] // _CHUNK[s])
        hs = _count_kernel(_NACC[s], _CHUNK[s], e_pad)(
            lidx_s, zeros128, ones128).reshape(2, _NACC[s], _D)
        hd = _count_kernel(_NACC[d], _CHUNK[d], e_pad)(
            lidx, zeros128, ones128).reshape(2, _NACC[d], _D)
        degs[et] = (
            (hs[0, :_NP[s], 0:1], hs[1, :_NP[s], 0:1]),
            (hd[0, :_NP[d], 0:1], hd[1, :_NP[d], 0:1]),
        )

    feats = fpad
    for l in range(2):
        per_dst = {t: [] for t in _NUM}
        for (s, et, d, ne) in _RELS:
            src, dst, lidx, e_pad = edges[et]
            (ds0, ds1), (dd0, dd1) = degs[et]
            w = inp["W%d_%s" % (l, et)]
            z = _transform(feats[s], ds0, ds1, w)
            m = _scatter_kernel(_NP[s], _NACC[d], _CHUNK[d], e_pad)(
                src, lidx, z, zeros128)
            m = m.reshape(2, _NACC[d], _D)
            per_dst[d].append(((m[0], m[1]), (dd0, dd1), et))
        new = {}
        for t in _NUM:
            ms = [p[0] for p in per_dst[t]]
            dds = [p[1] for p in per_dst[t]]
            bsum = sum(inp["b%d_%s" % (l, p[2])] for p in per_dst[t])
            bsum = bsum.reshape(1, _D)
            resid = fpad[t] if l == 1 else None
            new[t] = _combine(ms, dds, bsum, resid, _NP[t])
        feats = new

    return (feats["a"][:_NUM["a"]], feats["b"][:_NUM["b"]],
            feats["g"][:_NUM["g"]])
